# baseline (speedup 1.0000x reference)
import jax
import jax.numpy as jnp
from jax import lax
from jax.experimental import pallas as pl
from jax.experimental.pallas import tpu as pltpu
from jax.experimental.pallas import tpu_sc as plsc

N_LAYERS = 3
N = 10000
E = 320000
D = 128
NC = 2
NS = 16
F = D // NC
ROWS_PT = N // NS
RCHUNK = 125
ECHUNK = 128
CROWS = 2560
CPT = CROWS // NS
BLK = 16
NBLK = CPT // BLK
NRING = 4


def _bcast_lane(v16, e):
    idx = jnp.full((16, 1), e, dtype=jnp.int32)
    dnums = lax.GatherDimensionNumbers(
        offset_dims=(), collapsed_slice_dims=(0,), start_index_map=(0,))
    return lax.gather(v16, idx, dnums, (1,),
                      mode=lax.GatherScatterMode.PROMISE_IN_BOUNDS)


def _scatter_add(dst_ref, idx_ref, src_ref, sem):
    return pltpu.async_copy(src_ref, dst_ref.at[idx_ref], sem, add=True)


def _scale_unpack(gb, sb, val_row):
    @plsc.parallel_loop(0, ECHUNK, unroll=2)
    def _(e):
        e16 = (e // 16) * 16
        v16 = val_row[pl.ds(e16, 16)]
        vb = _bcast_lane(v16, e - e16)
        for j in range(F // 32):
            h = gb[e, pl.ds(j * 32, 32)]
            a0, a1 = plsc.unpack(h, format=plsc.PackFormat.INTERLEAVED)
            sb[e, pl.ds(j * 32, 16)] = a0 * vb
            sb[e, pl.ds(j * 32 + 16, 16)] = a1 * vb


def _pack_rows(src, dst):
    @plsc.parallel_loop(0, RCHUNK, unroll=1)
    def _(r):
        for j in range(F // 32):
            a0 = src[r, pl.ds(j * 32, 16)]
            a1 = src[r, pl.ds(j * 32 + 16, 16)]
            dst[r, pl.ds(j * 32, 32)] = plsc.pack(
                a0, a1, format=plsc.PackFormat.INTERLEAVED)


def _cib_offset(cib, nrows, cn):
    @plsc.parallel_loop(0, nrows * (ECHUNK // 16), unroll=2)
    def _(i):
        r = i // (ECHUNK // 16)
        j = i % (ECHUNK // 16)
        sl = pl.ds(j * 16, 16)
        cib[r, sl] = cib[r, sl] + cn


def _sc_body(rows_h, cols_h, vals_h, xflat_h, out_h, hcur_h,
             dst_sp, sb0, sb1, sb2, sb3, gb0, gb1, gb2, gb3, hb0, hb1,
             rib0, cib0, vvb0, rib1, cib1, vvb1,
             sem_i0, sem_i1, sem_z,
             sem_g0, sem_g1, sem_g2, sem_g3,
             sem_s0, sem_s1, sem_s2, sem_s3):
    c = lax.axis_index("c")
    s = lax.axis_index("s")
    cn = c * N
    row0 = s * ROWS_PT
    crow0 = s * CPT
    sbufs = (sb0, sb1, sb2, sb3)
    gbufs = (gb0, gb1, gb2, gb3)
    hbufs = (hb0, hb1)
    gsems = (sem_g0, sem_g1, sem_g2, sem_g3)
    ssems = (sem_s0, sem_s1, sem_s2, sem_s3)
    idxsets = ((rib0, cib0, vvb0, sem_i0), (rib1, cib1, vvb1, sem_i1))
    gs0 = sb0.at[pl.ds(0, RCHUNK), :]

    def idx_issue(blk, iset):
        rib, cib, vvb, sem = idxsets[iset]
        cbase = crow0 + blk * BLK
        pltpu.async_copy(rows_h.at[pl.ds(cbase, BLK), :], rib, sem)
        pltpu.async_copy(cols_h.at[pl.ds(cbase, BLK), :], cib, sem)
        pltpu.async_copy(vals_h.at[pl.ds(cbase, BLK), :], vvb, sem)

    def idx_wait(blk, iset):
        rib, cib, vvb, sem = idxsets[iset]
        cbase = crow0 + blk * BLK
        pltpu.make_async_copy(rows_h.at[pl.ds(cbase, BLK), :], rib, sem).wait()
        pltpu.make_async_copy(cols_h.at[pl.ds(cbase, BLK), :], cib, sem).wait()
        pltpu.make_async_copy(vals_h.at[pl.ds(cbase, BLK), :], vvb, sem).wait()
        _cib_offset(cib, BLK, cn)

    for k in range(ROWS_PT // RCHUNK):
        hsl = pl.ds(cn + row0 + k * RCHUNK, RCHUNK)
        pltpu.sync_copy(xflat_h.at[hsl, :], gs0)
        pltpu.sync_copy(gs0, out_h.at[hsl, :])
        _pack_rows(sb0, hb0)
        pltpu.sync_copy(hb0, hcur_h.at[hsl, :])

    @plsc.parallel_loop(0, RCHUNK * (F // 16), unroll=2)
    def _(i):
        r = i // (F // 16)
        j = i % (F // 16)
        sb0[r, pl.ds(j * 16, 16)] = jnp.zeros((16,), jnp.float32)
    zd = [pltpu.async_copy(
        gs0, dst_sp.at[pl.ds(row0 + k * RCHUNK, RCHUNK), :], sem_z)
        for k in range(ROWS_PT // RCHUNK)]
    for d in zd:
        d.wait()

    for layer in range(N_LAYERS):
        plsc.subcore_barrier()

        def run_block(blk, iset):
            rib, cib, vvb, _ = idxsets[iset]

            def gather(t, b):
                return pltpu.async_copy(
                    hcur_h.at[cib.at[t]], gbufs[b], gsems[b])

            dg = [None] * NRING
            dsx = [None] * NRING
            dg[0] = gather(0, 0)
            dg[1] = gather(1, 1)
            for t in range(BLK):
                b = t % NRING
                if t + 2 < BLK:
                    b2 = (t + 2) % NRING
                    dg[b2] = gather(t + 2, b2)
                dg[b].wait()
                if dsx[b] is not None:
                    dsx[b].wait()
                _scale_unpack(gbufs[b], sbufs[b], vvb.at[t])
                dsx[b] = _scatter_add(dst_sp, rib.at[t], sbufs[b], ssems[b])
            for b in range(NRING):
                if dsx[b] is not None:
                    dsx[b].wait()

        def pair(i, _):
            blkA = 2 * i
            idx_wait(blkA, 0)
            idx_issue(blkA + 1, 1)
            run_block(blkA, 0)
            idx_wait(blkA + 1, 1)

            @pl.when(i < NBLK // 2 - 1)
            def _():
                idx_issue(blkA + 2, 0)
            run_block(blkA + 1, 1)
            return 0

        idx_issue(0, 0)
        lax.fori_loop(0, NBLK // 2, pair, 0, unroll=False)

        plsc.subcore_barrier()

        wscale = (jnp.float32(1.0 / (N_LAYERS + 1))
                  if layer == N_LAYERS - 1 else jnp.float32(1.0))
        nk = ROWS_PT // RCHUNK
        wd = [None] * nk
        hd = [None] * nk
        zdl = [None] * nk
        rd = [None] * nk

        def issue_reads(k):
            ga, gb = (sb0, sb1) if k % 2 == 0 else (sb2, sb3)
            hsl = pl.ds(cn + row0 + k * RCHUNK, RCHUNK)
            ssl = pl.ds(row0 + k * RCHUNK, RCHUNK)
            return (pltpu.async_copy(dst_sp.at[ssl, :],
                                     ga.at[pl.ds(0, RCHUNK), :], sem_g0),
                    pltpu.async_copy(out_h.at[hsl, :],
                                     gb.at[pl.ds(0, RCHUNK), :], sem_g1))

        rd[0] = issue_reads(0)
        for k in range(nk):
            ga, gb = (sb0, sb1) if k % 2 == 0 else (sb2, sb3)
            hb = hbufs[k % 2]
            hsl = pl.ds(cn + row0 + k * RCHUNK, RCHUNK)
            ssl = pl.ds(row0 + k * RCHUNK, RCHUNK)
            if k + 1 < nk:
                for dlist in (wd, hd, zdl):
                    if dlist[k - 1] is not None:
                        dlist[k - 1].wait()
                        dlist[k - 1] = None
                rd[k + 1] = issue_reads(k + 1)
            da, db = rd[k]
            da.wait()
            if layer < N_LAYERS - 1:
                _pack_rows(ga, hb)
                hd[k] = pltpu.async_copy(hb, hcur_h.at[hsl, :], sem_z)
            db.wait()

            @plsc.parallel_loop(0, RCHUNK, unroll=2)
            def _(r):
                for j in range(F // 16):
                    sl = pl.ds(j * 16, 16)
                    gb[r, sl] = (gb[r, sl] + ga[r, sl]) * wscale
            wd[k] = pltpu.async_copy(gb.at[pl.ds(0, RCHUNK), :],
                                     out_h.at[hsl, :], sem_s1)
            if layer < N_LAYERS - 1:
                @plsc.parallel_loop(0, RCHUNK, unroll=4)
                def _(r):
                    for j in range(F // 16):
                        sl = pl.ds(j * 16, 16)
                        ga[r, sl] = jnp.zeros((16,), jnp.float32)
                zdl[k] = pltpu.async_copy(ga.at[pl.ds(0, RCHUNK), :],
                                          dst_sp.at[ssl, :], sem_s2)
        for dlist in (wd, hd, zdl):
            for d in dlist:
                if d is not None:
                    d.wait()


@jax.jit
def _lightgcn_sc(rows2, cols2, vals2, xflat):
    mesh = plsc.VectorSubcoreMesh(core_axis_name="c", subcore_axis_name="s",
                                  num_cores=NC, num_subcores=NS)
    out, _ = pl.kernel(
        _sc_body,
        out_type=(jax.ShapeDtypeStruct((NC * N, F), jnp.float32),
                  jax.ShapeDtypeStruct((NC * N, F), jnp.bfloat16)),
        mesh=mesh,
        scratch_types=[
            pltpu.VMEM_SHARED((N, F), jnp.float32),
            pltpu.VMEM((ECHUNK, F), jnp.float32),
            pltpu.VMEM((ECHUNK, F), jnp.float32),
            pltpu.VMEM((ECHUNK, F), jnp.float32),
            pltpu.VMEM((ECHUNK, F), jnp.float32),
            pltpu.VMEM((ECHUNK, F), jnp.bfloat16),
            pltpu.VMEM((ECHUNK, F), jnp.bfloat16),
            pltpu.VMEM((ECHUNK, F), jnp.bfloat16),
            pltpu.VMEM((ECHUNK, F), jnp.bfloat16),
            pltpu.VMEM((RCHUNK, F), jnp.bfloat16),
            pltpu.VMEM((RCHUNK, F), jnp.bfloat16),
            pltpu.VMEM((BLK, ECHUNK), jnp.int32),
            pltpu.VMEM((BLK, ECHUNK), jnp.int32),
            pltpu.VMEM((BLK, ECHUNK), jnp.float32),
            pltpu.VMEM((BLK, ECHUNK), jnp.int32),
            pltpu.VMEM((BLK, ECHUNK), jnp.int32),
            pltpu.VMEM((BLK, ECHUNK), jnp.float32),
            pltpu.SemaphoreType.DMA,
            pltpu.SemaphoreType.DMA,
            pltpu.SemaphoreType.DMA,
            pltpu.SemaphoreType.DMA,
            pltpu.SemaphoreType.DMA,
            pltpu.SemaphoreType.DMA,
            pltpu.SemaphoreType.DMA,
            pltpu.SemaphoreType.DMA,
            pltpu.SemaphoreType.DMA,
            pltpu.SemaphoreType.DMA,
            pltpu.SemaphoreType.DMA,
        ],
        compiler_params=pltpu.CompilerParams(use_tc_tiling_on_sc=False,
                                             needs_layout_passes=False),
        name="lightgcn_sc",
    )(rows2, cols2, vals2, xflat)
    return out


def kernel(norm_adj_edge_index, norm_adj_values, x):
    pad = CROWS * ECHUNK - E
    zi = jnp.zeros((pad,), jnp.int32)
    rows2 = jnp.concatenate([norm_adj_edge_index[0], zi]).reshape(CROWS,
                                                                  ECHUNK)
    cols2 = jnp.concatenate([norm_adj_edge_index[1], zi]).reshape(CROWS,
                                                                  ECHUNK)
    vals2 = jnp.concatenate(
        [norm_adj_values, jnp.zeros((pad,), jnp.float32)]).reshape(CROWS,
                                                                   ECHUNK)
    xflat = x.reshape(N, NC, F).transpose(1, 0, 2).reshape(NC * N, F)
    out = _lightgcn_sc(rows2, cols2, vals2, xflat)
    return out.reshape(NC, N, F).transpose(1, 0, 2).reshape(N, D)


# speedup vs baseline: 1.7325x; 1.7325x over previous
import jax
import jax.numpy as jnp
from jax import lax
from jax.experimental import pallas as pl
from jax.experimental.pallas import tpu as pltpu
from jax.experimental.pallas import tpu_sc as plsc

N_LAYERS = 3
N = 10000
E = 320000
D = 128
NC = 2
NS = 16
F = D // NC
ROWS_PT = N // NS
RCHUNK = 125
ECHUNK = 128
CROWS = 2560
CPT = CROWS // NS
BLK = 16
NBLK = CPT // BLK
NRING = 4


def _bcast_lane(v16, e):
    idx = jnp.full((16, 1), e, dtype=jnp.int32)
    dnums = lax.GatherDimensionNumbers(
        offset_dims=(), collapsed_slice_dims=(0,), start_index_map=(0,))
    return lax.gather(v16, idx, dnums, (1,),
                      mode=lax.GatherScatterMode.PROMISE_IN_BOUNDS)


def _scatter_add(dst_ref, idx_ref, src_ref, sem):
    return pltpu.async_copy(src_ref, dst_ref.at[idx_ref], sem, add=True)


def _scale_unpack(gb, sb, val_row):
    @plsc.parallel_loop(0, ECHUNK, unroll=2)
    def _(e):
        e16 = (e // 16) * 16
        v16 = val_row[pl.ds(e16, 16)]
        vb = _bcast_lane(v16, e - e16)
        for j in range(F // 32):
            h = gb[e, pl.ds(j * 32, 32)]
            a0, a1 = plsc.unpack(h, format=plsc.PackFormat.INTERLEAVED)
            sb[e, pl.ds(j * 32, 16)] = a0 * vb
            sb[e, pl.ds(j * 32 + 16, 16)] = a1 * vb


def _pack_rows(src, dst):
    @plsc.parallel_loop(0, RCHUNK, unroll=1)
    def _(r):
        for j in range(F // 32):
            a0 = src[r, pl.ds(j * 32, 16)]
            a1 = src[r, pl.ds(j * 32 + 16, 16)]
            dst[r, pl.ds(j * 32, 32)] = plsc.pack(
                a0, a1, format=plsc.PackFormat.INTERLEAVED)


def _cib_offset(cib, nrows, cn):
    @plsc.parallel_loop(0, nrows * (ECHUNK // 16), unroll=2)
    def _(i):
        r = i // (ECHUNK // 16)
        j = i % (ECHUNK // 16)
        sl = pl.ds(j * 16, 16)
        cib[r, sl] = cib[r, sl] + cn


def _sc_body(rows_h, cols_h, vals_h, xflat_h, out_h, hcur_h,
             dst_sp, sb0, sb1, sb2, sb3, gb0, gb1, gb2, gb3, hb0, hb1,
             rib0, cib0, vvb0, rib1, cib1, vvb1,
             sem_i0, sem_i1, sem_z,
             sem_g0, sem_g1, sem_g2, sem_g3,
             sem_s0, sem_s1, sem_s2, sem_s3):
    c = lax.axis_index("c")
    s = lax.axis_index("s")
    cn = c * N
    row0 = s * ROWS_PT
    crow0 = s * CPT
    sbufs = (sb0, sb1, sb2, sb3)
    gbufs = (gb0, gb1, gb2, gb3)
    hbufs = (hb0, hb1)
    gsems = (sem_g0, sem_g1, sem_g2, sem_g3)
    ssems = (sem_s0, sem_s1, sem_s2, sem_s3)
    idxsets = ((rib0, cib0, vvb0, sem_i0), (rib1, cib1, vvb1, sem_i1))
    gs0 = sb0.at[pl.ds(0, RCHUNK), :]

    def idx_issue(blk, iset):
        rib, cib, vvb, sem = idxsets[iset]
        cbase = crow0 + blk * BLK
        pltpu.async_copy(rows_h.at[pl.ds(cbase, BLK), :], rib, sem)
        pltpu.async_copy(cols_h.at[pl.ds(cbase, BLK), :], cib, sem)
        pltpu.async_copy(vals_h.at[pl.ds(cbase, BLK), :], vvb, sem)

    def idx_wait(blk, iset):
        rib, cib, vvb, sem = idxsets[iset]
        cbase = crow0 + blk * BLK
        pltpu.make_async_copy(rows_h.at[pl.ds(cbase, BLK), :], rib, sem).wait()
        pltpu.make_async_copy(cols_h.at[pl.ds(cbase, BLK), :], cib, sem).wait()
        pltpu.make_async_copy(vals_h.at[pl.ds(cbase, BLK), :], vvb, sem).wait()
        _cib_offset(cib, BLK, cn)

    for k in range(ROWS_PT // RCHUNK):
        hsl = pl.ds(cn + row0 + k * RCHUNK, RCHUNK)
        pltpu.sync_copy(xflat_h.at[hsl, :], gs0)
        pltpu.sync_copy(gs0, out_h.at[hsl, :])
        _pack_rows(sb0, hb0)
        pltpu.sync_copy(hb0, hcur_h.at[hsl, :])

    @plsc.parallel_loop(0, RCHUNK * (F // 16), unroll=2)
    def _(i):
        r = i // (F // 16)
        j = i % (F // 16)
        sb0[r, pl.ds(j * 16, 16)] = jnp.zeros((16,), jnp.float32)
    zd = [pltpu.async_copy(
        gs0, dst_sp.at[pl.ds(row0 + k * RCHUNK, RCHUNK), :], sem_z)
        for k in range(ROWS_PT // RCHUNK)]
    for d in zd:
        d.wait()

    for layer in range(N_LAYERS):
        plsc.subcore_barrier()

        def run_block(blk, iset):
            rib, cib, vvb, _ = idxsets[iset]

            def gather(t, b):
                return pltpu.async_copy(
                    hcur_h.at[cib.at[t]], gbufs[b], gsems[b])

            dg = [None] * NRING
            dsx = [None] * NRING
            dg[0] = gather(0, 0)
            dg[1] = gather(1, 1)
            for t in range(BLK):
                b = t % NRING
                if t + 2 < BLK:
                    b2 = (t + 2) % NRING
                    dg[b2] = gather(t + 2, b2)
                dg[b].wait()
                if dsx[b] is not None:
                    dsx[b].wait()
                _scale_unpack(gbufs[b], sbufs[b], vvb.at[t])
                dsx[b] = _scatter_add(dst_sp, rib.at[t], sbufs[b], ssems[b])
            for b in range(NRING):
                if dsx[b] is not None:
                    dsx[b].wait()

        def pair(i, _):
            blkA = 2 * i
            idx_wait(blkA, 0)
            idx_issue(blkA + 1, 1)
            run_block(blkA, 0)
            idx_wait(blkA + 1, 1)

            @pl.when(i < NBLK // 2 - 1)
            def _():
                idx_issue(blkA + 2, 0)
            run_block(blkA + 1, 1)
            return 0

        idx_issue(0, 0)
        lax.fori_loop(0, NBLK // 2, pair, 0, unroll=False)

        plsc.subcore_barrier()

        wscale = (jnp.float32(1.0 / (N_LAYERS + 1))
                  if layer == N_LAYERS - 1 else jnp.float32(1.0))
        nk = ROWS_PT // RCHUNK
        wd = [None] * nk
        hd = [None] * nk
        zdl = [None] * nk
        rd = [None] * nk

        def issue_reads(k):
            ga, gb = (sb0, sb1) if k % 2 == 0 else (sb2, sb3)
            hsl = pl.ds(cn + row0 + k * RCHUNK, RCHUNK)
            ssl = pl.ds(row0 + k * RCHUNK, RCHUNK)
            return (pltpu.async_copy(dst_sp.at[ssl, :],
                                     ga.at[pl.ds(0, RCHUNK), :], sem_g0),
                    pltpu.async_copy(out_h.at[hsl, :],
                                     gb.at[pl.ds(0, RCHUNK), :], sem_g1))

        rd[0] = issue_reads(0)
        for k in range(nk):
            ga, gb = (sb0, sb1) if k % 2 == 0 else (sb2, sb3)
            hb = hbufs[k % 2]
            hsl = pl.ds(cn + row0 + k * RCHUNK, RCHUNK)
            ssl = pl.ds(row0 + k * RCHUNK, RCHUNK)
            if k + 1 < nk:
                for dlist in (wd, hd, zdl):
                    if dlist[k - 1] is not None:
                        dlist[k - 1].wait()
                        dlist[k - 1] = None
                rd[k + 1] = issue_reads(k + 1)
            da, db = rd[k]
            da.wait()
            if layer < N_LAYERS - 1:
                _pack_rows(ga, hb)
                hd[k] = pltpu.async_copy(hb, hcur_h.at[hsl, :], sem_z)
            db.wait()

            @plsc.parallel_loop(0, RCHUNK, unroll=2)
            def _(r):
                for j in range(F // 16):
                    sl = pl.ds(j * 16, 16)
                    gb[r, sl] = (gb[r, sl] + ga[r, sl]) * wscale
            wd[k] = pltpu.async_copy(gb.at[pl.ds(0, RCHUNK), :],
                                     out_h.at[hsl, :], sem_s1)
            if layer < N_LAYERS - 1:
                @plsc.parallel_loop(0, RCHUNK, unroll=4)
                def _(r):
                    for j in range(F // 16):
                        sl = pl.ds(j * 16, 16)
                        ga[r, sl] = jnp.zeros((16,), jnp.float32)
                zdl[k] = pltpu.async_copy(ga.at[pl.ds(0, RCHUNK), :],
                                          dst_sp.at[ssl, :], sem_s2)
        for dlist in (wd, hd, zdl):
            for d in dlist:
                if d is not None:
                    d.wait()


@jax.jit
def _lightgcn_sc(rows2, cols2, vals2, xflat):
    mesh = plsc.VectorSubcoreMesh(core_axis_name="c", subcore_axis_name="s",
                                  num_cores=NC, num_subcores=NS)
    out, _ = pl.kernel(
        _sc_body,
        out_type=(jax.ShapeDtypeStruct((NC * N, F), jnp.float32),
                  jax.ShapeDtypeStruct((NC * N, F), jnp.bfloat16)),
        mesh=mesh,
        scratch_types=[
            pltpu.VMEM_SHARED((N, F), jnp.float32),
            pltpu.VMEM((ECHUNK, F), jnp.float32),
            pltpu.VMEM((ECHUNK, F), jnp.float32),
            pltpu.VMEM((ECHUNK, F), jnp.float32),
            pltpu.VMEM((ECHUNK, F), jnp.float32),
            pltpu.VMEM((ECHUNK, F), jnp.bfloat16),
            pltpu.VMEM((ECHUNK, F), jnp.bfloat16),
            pltpu.VMEM((ECHUNK, F), jnp.bfloat16),
            pltpu.VMEM((ECHUNK, F), jnp.bfloat16),
            pltpu.VMEM((RCHUNK, F), jnp.bfloat16),
            pltpu.VMEM((RCHUNK, F), jnp.bfloat16),
            pltpu.VMEM((BLK, ECHUNK), jnp.int32),
            pltpu.VMEM((BLK, ECHUNK), jnp.int32),
            pltpu.VMEM((BLK, ECHUNK), jnp.float32),
            pltpu.VMEM((BLK, ECHUNK), jnp.int32),
            pltpu.VMEM((BLK, ECHUNK), jnp.int32),
            pltpu.VMEM((BLK, ECHUNK), jnp.float32),
            pltpu.SemaphoreType.DMA,
            pltpu.SemaphoreType.DMA,
            pltpu.SemaphoreType.DMA,
            pltpu.SemaphoreType.DMA,
            pltpu.SemaphoreType.DMA,
            pltpu.SemaphoreType.DMA,
            pltpu.SemaphoreType.DMA,
            pltpu.SemaphoreType.DMA,
            pltpu.SemaphoreType.DMA,
            pltpu.SemaphoreType.DMA,
            pltpu.SemaphoreType.DMA,
        ],
        compiler_params=pltpu.CompilerParams(use_tc_tiling_on_sc=False,
                                             needs_layout_passes=False),
        name="lightgcn_sc",
    )(rows2, cols2, vals2, xflat)
    return out


def kernel(norm_adj_edge_index, norm_adj_values, x):
    pad = CROWS * ECHUNK - E
    zi = jnp.arange(pad, dtype=jnp.int32) % N
    rows2 = jnp.concatenate([norm_adj_edge_index[0], zi]).reshape(CROWS,
                                                                  ECHUNK)
    cols2 = jnp.concatenate([norm_adj_edge_index[1], zi]).reshape(CROWS,
                                                                  ECHUNK)
    vals2 = jnp.concatenate(
        [norm_adj_values, jnp.zeros((pad,), jnp.float32)]).reshape(CROWS,
                                                                   ECHUNK)
    xflat = x.reshape(N, NC, F).transpose(1, 0, 2).reshape(NC * N, F)
    out = _lightgcn_sc(rows2, cols2, vals2, xflat)
    return out.reshape(NC, N, F).transpose(1, 0, 2).reshape(N, D)


# speedup vs baseline: 1.7438x; 1.0065x over previous
import jax
import jax.numpy as jnp
from jax import lax
from jax.experimental import pallas as pl
from jax.experimental.pallas import tpu as pltpu
from jax.experimental.pallas import tpu_sc as plsc

N_LAYERS = 3
N = 10000
E = 320000
D = 128
NC = 2
NS = 16
F = D // NC
ROWS_PT = N // NS
RCHUNK = 125
ECHUNK = 128
CROWS = E // ECHUNK
CPT = CROWS // NS
EXTRA = CROWS - CPT * NS
BLK = 13
NBLK = CPT // BLK
NRING = 4


def _bcast_lane(v16, e):
    idx = jnp.full((16, 1), e, dtype=jnp.int32)
    dnums = lax.GatherDimensionNumbers(
        offset_dims=(), collapsed_slice_dims=(0,), start_index_map=(0,))
    return lax.gather(v16, idx, dnums, (1,),
                      mode=lax.GatherScatterMode.PROMISE_IN_BOUNDS)


def _scatter_add(dst_ref, idx_ref, src_ref, sem):
    return pltpu.async_copy(src_ref, dst_ref.at[idx_ref], sem, add=True)


def _scale_unpack(gb, sb, val_row):
    @plsc.parallel_loop(0, ECHUNK, unroll=2)
    def _(e):
        e16 = (e // 16) * 16
        v16 = val_row[pl.ds(e16, 16)]
        vb = _bcast_lane(v16, e - e16)
        for j in range(F // 32):
            h = gb[e, pl.ds(j * 32, 32)]
            a0, a1 = plsc.unpack(h, format=plsc.PackFormat.INTERLEAVED)
            sb[e, pl.ds(j * 32, 16)] = a0 * vb
            sb[e, pl.ds(j * 32 + 16, 16)] = a1 * vb


def _pack_rows(src, dst):
    @plsc.parallel_loop(0, RCHUNK, unroll=2)
    def _(r):
        for j in range(F // 32):
            a0 = src[r, pl.ds(j * 32, 16)]
            a1 = src[r, pl.ds(j * 32 + 16, 16)]
            dst[r, pl.ds(j * 32, 32)] = plsc.pack(
                a0, a1, format=plsc.PackFormat.INTERLEAVED)


def _cib_offset(cib, nrows, cn):
    @plsc.parallel_loop(0, nrows * (ECHUNK // 16), unroll=4)
    def _(i):
        r = i // (ECHUNK // 16)
        j = i % (ECHUNK // 16)
        sl = pl.ds(j * 16, 16)
        cib[r, sl] = cib[r, sl] + cn


def _sc_body(rows_h, cols_h, vals_h, xflat_h, out_h, hcur_h,
             dst_sp, sb0, sb1, sb2, sb3, gb0, gb1, gb2, gb3, hb0, hb1,
             rib0, cib0, vvb0, rib1, cib1, vvb1,
             sem_i0, sem_i1, sem_z,
             sem_g0, sem_g1, sem_g2, sem_g3,
             sem_s0, sem_s1, sem_s2, sem_s3):
    c = lax.axis_index("c")
    s = lax.axis_index("s")
    cn = c * N
    row0 = s * ROWS_PT
    crow0 = s * CPT
    sbufs = (sb0, sb1, sb2, sb3)
    gbufs = (gb0, gb1, gb2, gb3)
    hbufs = (hb0, hb1)
    gsems = (sem_g0, sem_g1, sem_g2, sem_g3)
    ssems = (sem_s0, sem_s1, sem_s2, sem_s3)
    idxsets = ((rib0, cib0, vvb0, sem_i0), (rib1, cib1, vvb1, sem_i1))
    gs0 = sb0.at[pl.ds(0, RCHUNK), :]

    def idx_issue(blk, iset):
        rib, cib, vvb, sem = idxsets[iset]
        cbase = crow0 + blk * BLK
        pltpu.async_copy(rows_h.at[pl.ds(cbase, BLK), :], rib, sem)
        pltpu.async_copy(cols_h.at[pl.ds(cbase, BLK), :], cib, sem)
        pltpu.async_copy(vals_h.at[pl.ds(cbase, BLK), :], vvb, sem)

    def idx_wait(blk, iset):
        rib, cib, vvb, sem = idxsets[iset]
        cbase = crow0 + blk * BLK
        pltpu.make_async_copy(rows_h.at[pl.ds(cbase, BLK), :], rib, sem).wait()
        pltpu.make_async_copy(cols_h.at[pl.ds(cbase, BLK), :], cib, sem).wait()
        pltpu.make_async_copy(vals_h.at[pl.ds(cbase, BLK), :], vvb, sem).wait()
        _cib_offset(cib, BLK, cn)

    for k in range(ROWS_PT // RCHUNK):
        hsl = pl.ds(cn + row0 + k * RCHUNK, RCHUNK)
        pltpu.sync_copy(xflat_h.at[hsl, :], gs0)
        pltpu.sync_copy(gs0, out_h.at[hsl, :])
        _pack_rows(sb0, hb0)
        pltpu.sync_copy(hb0, hcur_h.at[hsl, :])

    @plsc.parallel_loop(0, RCHUNK * (F // 16), unroll=4)
    def _(i):
        r = i // (F // 16)
        j = i % (F // 16)
        sb0[r, pl.ds(j * 16, 16)] = jnp.zeros((16,), jnp.float32)
    zd = [pltpu.async_copy(
        gs0, dst_sp.at[pl.ds(row0 + k * RCHUNK, RCHUNK), :], sem_z)
        for k in range(ROWS_PT // RCHUNK)]
    for d in zd:
        d.wait()

    for layer in range(N_LAYERS):
        plsc.subcore_barrier()

        def run_block(blk, iset):
            rib, cib, vvb, _ = idxsets[iset]

            def gather(t, b):
                return pltpu.async_copy(
                    hcur_h.at[cib.at[t]], gbufs[b], gsems[b])

            dg = [None] * NRING
            dsx = [None] * NRING
            dg[0] = gather(0, 0)
            dg[1] = gather(1, 1)
            for t in range(BLK):
                b = t % NRING
                if t + 2 < BLK:
                    b2 = (t + 2) % NRING
                    dg[b2] = gather(t + 2, b2)
                dg[b].wait()
                if dsx[b] is not None:
                    dsx[b].wait()
                _scale_unpack(gbufs[b], sbufs[b], vvb.at[t])
                dsx[b] = _scatter_add(dst_sp, rib.at[t], sbufs[b], ssems[b])
            for b in range(NRING):
                if dsx[b] is not None:
                    dsx[b].wait()

        def pair(i, _):
            blkA = 2 * i
            idx_wait(blkA, 0)
            idx_issue(blkA + 1, 1)
            run_block(blkA, 0)
            idx_wait(blkA + 1, 1)

            @pl.when(i < NBLK // 2 - 1)
            def _():
                idx_issue(blkA + 2, 0)
            run_block(blkA + 1, 1)
            return 0

        idx_issue(0, 0)
        lax.fori_loop(0, NBLK // 2, pair, 0, unroll=False)

        @pl.when(s < EXTRA)
        def _():
            cex = NS * CPT + s
            pltpu.sync_copy(rows_h.at[pl.ds(cex, 1), :],
                            rib0.at[pl.ds(0, 1), :])
            pltpu.sync_copy(cols_h.at[pl.ds(cex, 1), :],
                            cib0.at[pl.ds(0, 1), :])
            pltpu.sync_copy(vals_h.at[pl.ds(cex, 1), :],
                            vvb0.at[pl.ds(0, 1), :])
            _cib_offset(cib0, 1, cn)
            pltpu.sync_copy(hcur_h.at[cib0.at[0]], gb0)
            _scale_unpack(gb0, sb0, vvb0.at[0])
            _scatter_add(dst_sp, rib0.at[0], sb0, sem_s0).wait()

        plsc.subcore_barrier()

        wscale = (jnp.float32(1.0 / (N_LAYERS + 1))
                  if layer == N_LAYERS - 1 else jnp.float32(1.0))
        nk = ROWS_PT // RCHUNK
        wd = [None] * nk
        hd = [None] * nk
        zdl = [None] * nk
        rd = [None] * nk

        def issue_reads(k):
            ga, gb = (sb0, sb1) if k % 2 == 0 else (sb2, sb3)
            hsl = pl.ds(cn + row0 + k * RCHUNK, RCHUNK)
            ssl = pl.ds(row0 + k * RCHUNK, RCHUNK)
            return (pltpu.async_copy(dst_sp.at[ssl, :],
                                     ga.at[pl.ds(0, RCHUNK), :], sem_g0),
                    pltpu.async_copy(out_h.at[hsl, :],
                                     gb.at[pl.ds(0, RCHUNK), :], sem_g1))

        rd[0] = issue_reads(0)
        for k in range(nk):
            ga, gb = (sb0, sb1) if k % 2 == 0 else (sb2, sb3)
            hb = hbufs[k % 2]
            hsl = pl.ds(cn + row0 + k * RCHUNK, RCHUNK)
            ssl = pl.ds(row0 + k * RCHUNK, RCHUNK)
            if k + 1 < nk:
                for dlist in (wd, hd, zdl):
                    if dlist[k - 1] is not None:
                        dlist[k - 1].wait()
                        dlist[k - 1] = None
                rd[k + 1] = issue_reads(k + 1)
            da, db = rd[k]
            da.wait()
            if layer < N_LAYERS - 1:
                _pack_rows(ga, hb)
                hd[k] = pltpu.async_copy(hb, hcur_h.at[hsl, :], sem_z)
            db.wait()

            @plsc.parallel_loop(0, RCHUNK, unroll=2)
            def _(r):
                for j in range(F // 16):
                    sl = pl.ds(j * 16, 16)
                    gb[r, sl] = (gb[r, sl] + ga[r, sl]) * wscale
            wd[k] = pltpu.async_copy(gb.at[pl.ds(0, RCHUNK), :],
                                     out_h.at[hsl, :], sem_s1)
            if layer < N_LAYERS - 1:
                @plsc.parallel_loop(0, RCHUNK, unroll=4)
                def _(r):
                    for j in range(F // 16):
                        sl = pl.ds(j * 16, 16)
                        ga[r, sl] = jnp.zeros((16,), jnp.float32)
                zdl[k] = pltpu.async_copy(ga.at[pl.ds(0, RCHUNK), :],
                                          dst_sp.at[ssl, :], sem_s2)
        for dlist in (wd, hd, zdl):
            for d in dlist:
                if d is not None:
                    d.wait()


@jax.jit
def _lightgcn_sc(rows2, cols2, vals2, xflat):
    mesh = plsc.VectorSubcoreMesh(core_axis_name="c", subcore_axis_name="s",
                                  num_cores=NC, num_subcores=NS)
    out, _ = pl.kernel(
        _sc_body,
        out_type=(jax.ShapeDtypeStruct((NC * N, F), jnp.float32),
                  jax.ShapeDtypeStruct((NC * N, F), jnp.bfloat16)),
        mesh=mesh,
        scratch_types=[
            pltpu.VMEM_SHARED((N, F), jnp.float32),
            pltpu.VMEM((ECHUNK, F), jnp.float32),
            pltpu.VMEM((ECHUNK, F), jnp.float32),
            pltpu.VMEM((ECHUNK, F), jnp.float32),
            pltpu.VMEM((ECHUNK, F), jnp.float32),
            pltpu.VMEM((ECHUNK, F), jnp.bfloat16),
            pltpu.VMEM((ECHUNK, F), jnp.bfloat16),
            pltpu.VMEM((ECHUNK, F), jnp.bfloat16),
            pltpu.VMEM((ECHUNK, F), jnp.bfloat16),
            pltpu.VMEM((RCHUNK, F), jnp.bfloat16),
            pltpu.VMEM((RCHUNK, F), jnp.bfloat16),
            pltpu.VMEM((BLK, ECHUNK), jnp.int32),
            pltpu.VMEM((BLK, ECHUNK), jnp.int32),
            pltpu.VMEM((BLK, ECHUNK), jnp.float32),
            pltpu.VMEM((BLK, ECHUNK), jnp.int32),
            pltpu.VMEM((BLK, ECHUNK), jnp.int32),
            pltpu.VMEM((BLK, ECHUNK), jnp.float32),
            pltpu.SemaphoreType.DMA,
            pltpu.SemaphoreType.DMA,
            pltpu.SemaphoreType.DMA,
            pltpu.SemaphoreType.DMA,
            pltpu.SemaphoreType.DMA,
            pltpu.SemaphoreType.DMA,
            pltpu.SemaphoreType.DMA,
            pltpu.SemaphoreType.DMA,
            pltpu.SemaphoreType.DMA,
            pltpu.SemaphoreType.DMA,
            pltpu.SemaphoreType.DMA,
        ],
        compiler_params=pltpu.CompilerParams(use_tc_tiling_on_sc=False,
                                             needs_layout_passes=False),
        name="lightgcn_sc",
    )(rows2, cols2, vals2, xflat)
    return out


def kernel(norm_adj_edge_index, norm_adj_values, x):
    rows2 = norm_adj_edge_index[0].reshape(CROWS, ECHUNK)
    cols2 = norm_adj_edge_index[1].reshape(CROWS, ECHUNK)
    vals2 = norm_adj_values.reshape(CROWS, ECHUNK)
    xflat = x.reshape(N, NC, F).transpose(1, 0, 2).reshape(NC * N, F)
    out = _lightgcn_sc(rows2, cols2, vals2, xflat)
    return out.reshape(NC, N, F).transpose(1, 0, 2).reshape(N, D)
